# baseline (speedup 1.0000x reference)
import jax
import jax.numpy as jnp
from jax.experimental import pallas as pl
from jax.experimental.pallas import tpu as pltpu


def _pool_flat(x, k):
    c, ho, kw = x.shape
    w = kw // k
    while kw > w:
        kw //= 2
        x = jnp.maximum(x[..., :kw], x[..., kw:2 * kw])
    s = 1
    while s < k:
        x = jnp.maximum(x, jnp.roll(x, -s, axis=-1))
        s *= 2
    x = x.reshape(c * ho, w)
    cw = min(w, 128)
    idx = jax.lax.broadcasted_iota(jnp.int32, (c * ho, cw // k), 1) * k
    parts = [jnp.take_along_axis(x[:, o:o + cw], idx, axis=1)
             for o in range(0, w, cw)]
    x = parts[0] if len(parts) == 1 else jnp.concatenate(parts, axis=1)
    return x.reshape(c, ho, w // k)


def _pool_body(x2_ref, x3_ref, x4_ref, p2_ref, p3_ref, p4_ref):
    p2_ref[0] = _pool_flat(x2_ref[0], 4)
    p3_ref[0] = _pool_flat(x3_ref[0], 8)
    p4_ref[0] = _pool_flat(x4_ref[0], 16)


def _main_body(x1_ref, p2_ref, p3_ref, p4_ref, ff_ref, out_ref):
    p1 = _pool_flat(x1_ref[0, 0, 0], 2)
    base = p1 + p2_ref[0] + p3_ref[0] + p4_ref[0]
    out_ref[0, 0, 0, 0] = jnp.maximum(base + ff_ref[0, 0, 0, 0], 0.0)
    out_ref[0, 1, 0, 0] = jnp.maximum(base + ff_ref[0, 1, 0, 0], 0.0)


def kernel(x1, x2, x3, x4, pure_ff):
    f32 = jnp.float32
    b = x1.shape[0]

    p2, p3, p4 = pl.pallas_call(
        _pool_body,
        grid=(b, 16),
        in_specs=[
            pl.BlockSpec((1, 256, 16, 256), lambda i, j: (i, j, 0, 0)),
            pl.BlockSpec((1, 16, 16, 1024), lambda i, j: (i, j, 0, 0)),
            pl.BlockSpec((1, 1, 16, 4096), lambda i, j: (i, 0, 0, 0)),
        ],
        out_specs=[
            pl.BlockSpec((1, 256, 16, 16), lambda i, j: (i, j, 0, 0)),
            pl.BlockSpec((1, 16, 16, 16), lambda i, j: (i, j, 0, 0)),
            pl.BlockSpec((1, 1, 16, 16), lambda i, j: (i, 0, 0, 0)),
        ],
        out_shape=[
            jax.ShapeDtypeStruct((b, 4096, 16, 16), f32),
            jax.ShapeDtypeStruct((b, 256, 16, 16), f32),
            jax.ShapeDtypeStruct((b, 1, 16, 16), f32),
        ],
        compiler_params=pltpu.CompilerParams(
            dimension_semantics=("parallel", "arbitrary")),
    )(x2.reshape(b, 4096, 16, 256),
      x3.reshape(b, 256, 16, 1024),
      x4.reshape(b, 1, 16, 4096))

    x1v = x1.reshape(b, 4, 16, 256, 16, 64)
    ffv = pure_ff.reshape(b, 2, 4, 16, 256, 16, 16)

    outv = pl.pallas_call(
        _main_body,
        grid=(b, 16, 4),
        in_specs=[
            pl.BlockSpec((1, 1, 1, 256, 16, 64),
                         lambda i, j, q: (i, q, j, 0, 0, 0)),
            pl.BlockSpec((1, 256, 16, 16), lambda i, j, q: (i, j, 0, 0)),
            pl.BlockSpec((1, 256, 16, 16), lambda i, j, q: (i, 0, 0, 0)),
            pl.BlockSpec((1, 1, 16, 16), lambda i, j, q: (i, 0, 0, 0)),
            pl.BlockSpec((1, 2, 1, 1, 256, 16, 16),
                         lambda i, j, q: (i, 0, q, j, 0, 0, 0)),
        ],
        out_specs=pl.BlockSpec((1, 2, 1, 1, 256, 16, 16),
                               lambda i, j, q: (i, 0, q, j, 0, 0, 0)),
        out_shape=jax.ShapeDtypeStruct((b, 2, 4, 16, 256, 16, 16), f32),
        compiler_params=pltpu.CompilerParams(
            dimension_semantics=("parallel", "arbitrary", "arbitrary")),
    )(x1v, p2, p3, p4, ffv)

    return outv.reshape(b, 32768, 16, 16)
